# baseline (speedup 1.0000x reference)
import functools

import jax
import jax.numpy as jnp
from jax import lax
from jax.experimental import pallas as pl
from jax.experimental.pallas import tpu as pltpu
from jax.experimental.pallas import tpu_sc as plsc

N_FIELDS = 26
VOCAB = 100000
EMB = 16
BATCH = 4096
D = N_FIELDS * EMB
HID = 256
N_UNITS = 3

NC = 2
NS = 16
NW = NC * NS
BPW = BATCH // NW


def _sc_gather(tables, idx_t):
    mesh = plsc.VectorSubcoreMesh(core_axis_name="c", subcore_axis_name="s")

    @functools.partial(
        pl.kernel,
        out_type=jax.ShapeDtypeStruct((BATCH, D), jnp.float32),
        mesh=mesh,
        scratch_types=[
            pltpu.VMEM((N_FIELDS, BPW), jnp.int32),
            pltpu.VMEM((N_FIELDS, BPW, EMB), jnp.float32),
            pltpu.SemaphoreType.DMA,
        ],
        compiler_params=pltpu.CompilerParams(use_tc_tiling_on_sc=False),
    )
    def gather_kernel(tbl_hbm, idx_hbm, out_hbm, idx_v, rows_v, sem):
        wid = lax.axis_index("s") * NC + lax.axis_index("c")
        b0 = wid * BPW
        pltpu.sync_copy(idx_hbm.at[:, pl.ds(b0, BPW)], idx_v)
        copies = []
        for f in range(N_FIELDS):
            copies.append(
                pltpu.async_copy(
                    tbl_hbm.at[f].at[idx_v.at[f]], rows_v.at[f], sem
                )
            )
        for c in copies:
            c.wait()
        for f in range(N_FIELDS):
            pltpu.sync_copy(
                rows_v.at[f],
                out_hbm.at[pl.ds(b0, BPW), pl.ds(f * EMB, EMB)],
            )

    return gather_kernel(tables, idx_t)


def _mlp_body(x_ref, w1_ref, b1_ref, w2_ref, b2_ref, wd_ref, bd_ref, o_ref):
    r = x_ref[...]
    for i in range(N_UNITS):
        h = jnp.dot(r, w1_ref[i], preferred_element_type=jnp.float32)
        h = jnp.maximum(h + b1_ref[i], 0.0)
        h = jnp.dot(h, w2_ref[i], preferred_element_type=jnp.float32)
        r = jnp.maximum(h + b2_ref[i] + r, 0.0)
    z = jnp.sum(r * wd_ref[...], axis=1, keepdims=True) + bd_ref[...]
    o_ref[...] = 1.0 / (1.0 + jnp.exp(-z))


def _mlp(emb, res_W1, res_b1, res_W2, res_b2, wd_row, bd11, block_b=512):
    grid = (BATCH // block_b,)
    return pl.pallas_call(
        _mlp_body,
        grid=grid,
        in_specs=[
            pl.BlockSpec((block_b, D), lambda i: (i, 0)),
            pl.BlockSpec((N_UNITS, D, HID), lambda i: (0, 0, 0)),
            pl.BlockSpec((N_UNITS, 1, HID), lambda i: (0, 0, 0)),
            pl.BlockSpec((N_UNITS, HID, D), lambda i: (0, 0, 0)),
            pl.BlockSpec((N_UNITS, 1, D), lambda i: (0, 0, 0)),
            pl.BlockSpec((1, D), lambda i: (0, 0)),
            pl.BlockSpec((1, 1), lambda i: (0, 0)),
        ],
        out_specs=pl.BlockSpec((block_b, 1), lambda i: (i, 0)),
        out_shape=jax.ShapeDtypeStruct((BATCH, 1), jnp.float32),
    )(emb, res_W1, res_b1, res_W2, res_b2, wd_row, bd11)


def kernel(inputs, tables, res_W1, res_b1, res_W2, res_b2, Wd, bd):
    idx_t = inputs.astype(jnp.int32).T
    emb = _sc_gather(tables, idx_t)
    out = _mlp(
        emb,
        res_W1,
        res_b1.reshape(N_UNITS, 1, HID),
        res_W2,
        res_b2.reshape(N_UNITS, 1, D),
        Wd.reshape(1, D),
        bd.reshape(1, 1),
    )
    return out


# speedup vs baseline: 2.9666x; 2.9666x over previous
import functools

import jax
import jax.numpy as jnp
from jax import lax
from jax.experimental import pallas as pl
from jax.experimental.pallas import tpu as pltpu
from jax.experimental.pallas import tpu_sc as plsc

N_FIELDS = 26
VOCAB = 100000
EMB = 16
BATCH = 4096
D = N_FIELDS * EMB
HID = 256
N_UNITS = 3

NC = 2
NS = 16
WIN = 1024
NWIN = 98
LAST_CW = 640
TAIL0 = 97 * WIN + LAST_CW
TAILW = VOCAB - TAIL0
NBK = 13
ACC_W = BATCH + 1
SENT_B = BATCH


def _sc_gather(tt, tail, keys2):
    mesh = plsc.VectorSubcoreMesh(core_axis_name="c", subcore_axis_name="s")

    @functools.partial(
        pl.kernel,
        out_type=jax.ShapeDtypeStruct((D, BATCH), jnp.float32),
        mesh=mesh,
        scratch_types=[
            pltpu.VMEM((EMB, WIN), jnp.float32),
            pltpu.VMEM((EMB, TAILW), jnp.float32),
            pltpu.VMEM((32, 128), jnp.int32),
            pltpu.VMEM((BATCH + 16,), jnp.int32),
            pltpu.VMEM((BATCH + 16,), jnp.int32),
            pltpu.VMEM((EMB, ACC_W), jnp.float32),
        ],
        compiler_params=pltpu.CompilerParams(needs_layout_passes=False),
    )
    def gather_kernel(tt_hbm, tail_hbm, keys_hbm, out_hbm,
                      chunk_v, tail_v, keys_v, bkt_v, hits_v, acc_v):
        f = lax.axis_index("s") * NC + lax.axis_index("c")
        row_iota = lax.iota(jnp.int32, 16)

        def compact(dst, kv, m, cnt):
            mi = m.astype(jnp.int32)
            pos = cnt + plsc.cumsum(mi) - mi
            plsc.store_scatter(dst, [pos], kv, mask=m)
            return cnt + plsc.all_reduce_population_count(m)[0]

        @pl.when(f < N_FIELDS)
        def _():
            fr = pl.multiple_of(f * 32, 8)
            fe = pl.multiple_of(f * EMB, 8)
            pltpu.sync_copy(keys_hbm.at[pl.ds(fr, 32)], keys_v)
            pltpu.sync_copy(tail_hbm.at[pl.ds(fe, EMB), :], tail_v)

            def bucket_body(bk, _c0):
                blo = bk << 26
                bhi = (bk + 1) << 26

                def bscan(g, cnt):
                    for cc in range(8):
                        kv = keys_v[g, pl.ds(cc * 16, 16)]
                        m = (kv >= blo) & (kv < bhi)
                        cnt = compact(bkt_v, kv, m, cnt)
                    return cnt

                nbk = lax.fori_loop(0, 32, bscan, 0)
                plsc.store_scatter(
                    bkt_v, [nbk + row_iota],
                    jnp.full((16,), -1, jnp.int32),
                )

                def win_body(wi, _c1):
                    w = bk * 8 + wi

                    @pl.when(w < NWIN)
                    def _():
                        w0 = w * WIN
                        w0a = pl.multiple_of(w0, 128)

                        @pl.when(w < NWIN - 1)
                        def _():
                            pltpu.sync_copy(
                                tt_hbm.at[pl.ds(fe, EMB), pl.ds(w0a, WIN)],
                                chunk_v,
                            )

                        @pl.when(w == NWIN - 1)
                        def _():
                            pltpu.sync_copy(
                                tt_hbm.at[pl.ds(fe, EMB), pl.ds(w0a, LAST_CW)],
                                chunk_v.at[:, pl.ds(0, LAST_CW)],
                            )

                        lo = w0 << 13
                        hi = (w0 + WIN) << 13

                        def wscan(g, cnt):
                            kv = plsc.load_gather(bkt_v, [g * 16 + row_iota])
                            m = (kv >= lo) & (kv < hi)
                            return compact(hits_v, kv, m, cnt)

                        ngb = (nbk + 15) // 16
                        cnt = lax.fori_loop(0, ngb, wscan, 0)
                        sent = jnp.full(
                            (16,), (w0 << 13) | SENT_B, dtype=jnp.int32
                        )
                        plsc.store_scatter(hits_v, [cnt + row_iota], sent)

                        def hit_group(j, _c2):
                            kv = plsc.load_gather(
                                hits_v, [j * 16 + row_iota]
                            )
                            for l in range(16):
                                key = kv[l]
                                v = key >> 13
                                b = key & 8191
                                col = v - w0
                                bcol = jnp.full((16,), b, jnp.int32)

                                @pl.when(v < TAIL0)
                                def _():
                                    cv = plsc.load_gather(
                                        chunk_v,
                                        [row_iota,
                                         jnp.full((16,), col, jnp.int32)],
                                    )
                                    plsc.store_scatter(
                                        acc_v, [row_iota, bcol], cv
                                    )

                                @pl.when(v >= TAIL0)
                                def _():
                                    cv = plsc.load_gather(
                                        tail_v,
                                        [row_iota,
                                         jnp.full((16,), v - TAIL0,
                                                  jnp.int32)],
                                    )
                                    plsc.store_scatter(
                                        acc_v, [row_iota, bcol], cv
                                    )
                            return _c2

                        lax.fori_loop(0, (cnt + 15) // 16, hit_group, 0)

                    return _c1

                lax.fori_loop(0, 8, win_body, 0)
                return _c0

            lax.fori_loop(0, NBK, bucket_body, 0)
            pltpu.sync_copy(
                acc_v.at[:, pl.ds(0, BATCH)], out_hbm.at[pl.ds(fe, EMB), :]
            )

    return gather_kernel(tt, tail, keys2)


def _mlp_body(x_ref, w1_ref, b1_ref, w2_ref, b2_ref, wd_ref, bd_ref, o_ref):
    r = x_ref[...].T
    for i in range(N_UNITS):
        h = jnp.dot(r, w1_ref[i], preferred_element_type=jnp.float32)
        h = jnp.maximum(h + b1_ref[i], 0.0)
        h = jnp.dot(h, w2_ref[i], preferred_element_type=jnp.float32)
        r = jnp.maximum(h + b2_ref[i] + r, 0.0)
    z = jnp.sum(r * wd_ref[...], axis=1, keepdims=True) + bd_ref[...]
    o_ref[...] = 1.0 / (1.0 + jnp.exp(-z))


def _mlp(emb_t, res_W1, res_b1, res_W2, res_b2, wd_row, bd11, block_b=512):
    grid = (BATCH // block_b,)
    return pl.pallas_call(
        _mlp_body,
        grid=grid,
        in_specs=[
            pl.BlockSpec((D, block_b), lambda i: (0, i)),
            pl.BlockSpec((N_UNITS, D, HID), lambda i: (0, 0, 0)),
            pl.BlockSpec((N_UNITS, 1, HID), lambda i: (0, 0, 0)),
            pl.BlockSpec((N_UNITS, HID, D), lambda i: (0, 0, 0)),
            pl.BlockSpec((N_UNITS, 1, D), lambda i: (0, 0, 0)),
            pl.BlockSpec((1, D), lambda i: (0, 0)),
            pl.BlockSpec((1, 1), lambda i: (0, 0)),
        ],
        out_specs=pl.BlockSpec((block_b, 1), lambda i: (i, 0)),
        out_shape=jax.ShapeDtypeStruct((BATCH, 1), jnp.float32),
    )(emb_t, res_W1, res_b1, res_W2, res_b2, wd_row, bd11)


def kernel(inputs, tables, res_W1, res_b1, res_W2, res_b2, Wd, bd):
    tt = tables.transpose(0, 2, 1).reshape(D, VOCAB)
    tail = lax.slice(tt, (0, TAIL0), (D, VOCAB))
    keys2 = (
        (inputs.astype(jnp.int32).T << 13)
        | jnp.arange(BATCH, dtype=jnp.int32)[None, :]
    ).reshape(832, 128)
    emb_t = _sc_gather(tt, tail, keys2)
    out = _mlp(
        emb_t,
        res_W1,
        res_b1.reshape(N_UNITS, 1, HID),
        res_W2,
        res_b2.reshape(N_UNITS, 1, D),
        Wd.reshape(1, D),
        bd.reshape(1, 1),
    )
    return out


# speedup vs baseline: 4.3967x; 1.4821x over previous
import functools

import jax
import jax.numpy as jnp
from jax import lax
from jax.experimental import pallas as pl
from jax.experimental.pallas import tpu as pltpu
from jax.experimental.pallas import tpu_sc as plsc

N_FIELDS = 26
VOCAB = 100000
EMB = 16
BATCH = 4096
D = N_FIELDS * EMB
HID = 256
N_UNITS = 3

NC = 2
NS = 16
WIN = 1024
NWIN = 98
LAST_CW = 640
TAIL0 = 97 * WIN + LAST_CW
TAILW = VOCAB - TAIL0
NBK = 13
ACC_W = BATCH + 1
SENT_B = BATCH


def _sc_gather(tt, tail, keys2):
    mesh = plsc.VectorSubcoreMesh(core_axis_name="c", subcore_axis_name="s")

    @functools.partial(
        pl.kernel,
        out_type=jax.ShapeDtypeStruct((D, BATCH), jnp.float32),
        mesh=mesh,
        scratch_types=[
            pltpu.VMEM((2, EMB, WIN), jnp.float32),
            pltpu.VMEM((EMB, TAILW), jnp.float32),
            pltpu.VMEM((32, 128), jnp.int32),
            pltpu.VMEM((BATCH + 16,), jnp.int32),
            pltpu.VMEM((BATCH + 16,), jnp.int32),
            pltpu.VMEM((EMB, ACC_W), jnp.float32),
            pltpu.SemaphoreType.DMA,
        ],
        compiler_params=pltpu.CompilerParams(needs_layout_passes=False),
    )
    def gather_kernel(tt_hbm, tail_hbm, keys_hbm, out_hbm,
                      chunk_v, tail_v, keys_v, bkt_v, hits_v, acc_v, csem):
        f = lax.axis_index("s") * NC + lax.axis_index("c")
        row_iota = lax.iota(jnp.int32, 16)

        def compact(dst, kv, m, cnt):
            mi = m.astype(jnp.int32)
            incl = plsc.cumsum(mi)
            plsc.store_scatter(dst, [cnt + incl - mi], kv, mask=m)
            return cnt + incl[15]

        @pl.when(f < N_FIELDS)
        def _():
            fr = pl.multiple_of(f * 32, 8)
            fe = pl.multiple_of(f * EMB, 8)
            pltpu.sync_copy(keys_hbm.at[pl.ds(fr, 32)], keys_v)
            pltpu.sync_copy(tail_hbm.at[pl.ds(fe, EMB), :], tail_v)

            def fire(w, p):
                w0a = pl.multiple_of(w * WIN, 128)

                @pl.when(w < NWIN - 1)
                def _():
                    pltpu.async_copy(
                        tt_hbm.at[pl.ds(fe, EMB), pl.ds(w0a, WIN)],
                        chunk_v.at[p], csem,
                    )

                @pl.when(w == NWIN - 1)
                def _():
                    pltpu.async_copy(
                        tt_hbm.at[pl.ds(fe, EMB), pl.ds(w0a, LAST_CW)],
                        chunk_v.at[p].at[:, pl.ds(0, LAST_CW)], csem,
                    )

            def wait_chunk(w, p):
                @pl.when(w < NWIN - 1)
                def _():
                    pltpu.make_async_copy(
                        tt_hbm.at[pl.ds(fe, EMB), pl.ds(0, WIN)],
                        chunk_v.at[p], csem,
                    ).wait()

                @pl.when(w == NWIN - 1)
                def _():
                    pltpu.make_async_copy(
                        tt_hbm.at[pl.ds(fe, EMB), pl.ds(0, LAST_CW)],
                        chunk_v.at[p].at[:, pl.ds(0, LAST_CW)], csem,
                    ).wait()

            fire(0, 0)

            def bucket_body(bk, _c0):
                blo = bk << 26
                bhi = (bk + 1) << 26

                def bscan(g, cnt):
                    for cc in range(8):
                        kv = keys_v[g, pl.ds(cc * 16, 16)]
                        m = (kv >= blo) & (kv < bhi)
                        cnt = compact(bkt_v, kv, m, cnt)
                    return cnt

                nbk = lax.fori_loop(0, 32, bscan, 0)
                plsc.store_scatter(
                    bkt_v, [nbk + row_iota],
                    jnp.full((16,), -1, jnp.int32),
                )

                def win_body(wi, _c1):
                    w = bk * 8 + wi

                    @pl.when(w < NWIN)
                    def _():
                        w0 = w * WIN
                        p = w % 2
                        chk = chunk_v.at[p]
                        wait_chunk(w, p)

                        @pl.when(w + 1 < NWIN)
                        def _():
                            fire(w + 1, 1 - p)

                        lo = w0 << 13
                        hi = (w0 + WIN) << 13

                        def wscan(g, cnt):
                            kv = plsc.load_gather(bkt_v, [g * 16 + row_iota])
                            m = (kv >= lo) & (kv < hi)
                            return compact(hits_v, kv, m, cnt)

                        ngb = (nbk + 15) // 16
                        cnt = lax.fori_loop(0, ngb, wscan, 0)
                        sent = jnp.full(
                            (16,), (w0 << 13) | SENT_B, dtype=jnp.int32
                        )
                        plsc.store_scatter(hits_v, [cnt + row_iota], sent)

                        def hit_group(j, _c2):
                            kv = plsc.load_gather(
                                hits_v, [j * 16 + row_iota]
                            )
                            for l in range(16):
                                key = kv[l]
                                v = key >> 13
                                b = key & 8191
                                col = v - w0
                                bcol = jnp.full((16,), b, jnp.int32)

                                @pl.when(v < TAIL0)
                                def _():
                                    cv = plsc.load_gather(
                                        chk,
                                        [row_iota,
                                         jnp.full((16,), col, jnp.int32)],
                                    )
                                    plsc.store_scatter(
                                        acc_v, [row_iota, bcol], cv
                                    )

                                @pl.when(v >= TAIL0)
                                def _():
                                    cv = plsc.load_gather(
                                        tail_v,
                                        [row_iota,
                                         jnp.full((16,), v - TAIL0,
                                                  jnp.int32)],
                                    )
                                    plsc.store_scatter(
                                        acc_v, [row_iota, bcol], cv
                                    )
                            return _c2

                        lax.fori_loop(0, (cnt + 15) // 16, hit_group, 0)

                    return _c1

                lax.fori_loop(0, 8, win_body, 0)
                return _c0

            lax.fori_loop(0, NBK, bucket_body, 0)
            pltpu.sync_copy(
                acc_v.at[:, pl.ds(0, BATCH)], out_hbm.at[pl.ds(fe, EMB), :]
            )

    return gather_kernel(tt, tail, keys2)


def _mlp_body(x_ref, w1_ref, b1_ref, w2_ref, b2_ref, wd_ref, bd_ref, o_ref):
    r = x_ref[...].T
    for i in range(N_UNITS):
        h = jnp.dot(r, w1_ref[i], preferred_element_type=jnp.float32)
        h = jnp.maximum(h + b1_ref[i], 0.0)
        h = jnp.dot(h, w2_ref[i], preferred_element_type=jnp.float32)
        r = jnp.maximum(h + b2_ref[i] + r, 0.0)
    z = jnp.sum(r * wd_ref[...], axis=1, keepdims=True) + bd_ref[...]
    o_ref[...] = 1.0 / (1.0 + jnp.exp(-z))


def _mlp(emb_t, res_W1, res_b1, res_W2, res_b2, wd_row, bd11, block_b=512):
    grid = (BATCH // block_b,)
    return pl.pallas_call(
        _mlp_body,
        grid=grid,
        in_specs=[
            pl.BlockSpec((D, block_b), lambda i: (0, i)),
            pl.BlockSpec((N_UNITS, D, HID), lambda i: (0, 0, 0)),
            pl.BlockSpec((N_UNITS, 1, HID), lambda i: (0, 0, 0)),
            pl.BlockSpec((N_UNITS, HID, D), lambda i: (0, 0, 0)),
            pl.BlockSpec((N_UNITS, 1, D), lambda i: (0, 0, 0)),
            pl.BlockSpec((1, D), lambda i: (0, 0)),
            pl.BlockSpec((1, 1), lambda i: (0, 0)),
        ],
        out_specs=pl.BlockSpec((block_b, 1), lambda i: (i, 0)),
        out_shape=jax.ShapeDtypeStruct((BATCH, 1), jnp.float32),
    )(emb_t, res_W1, res_b1, res_W2, res_b2, wd_row, bd11)


def kernel(inputs, tables, res_W1, res_b1, res_W2, res_b2, Wd, bd):
    tt = tables.transpose(0, 2, 1).reshape(D, VOCAB)
    tail = lax.slice(tt, (0, TAIL0), (D, VOCAB))
    keys2 = (
        (inputs.astype(jnp.int32).T << 13)
        | jnp.arange(BATCH, dtype=jnp.int32)[None, :]
    ).reshape(832, 128)
    emb_t = _sc_gather(tt, tail, keys2)
    out = _mlp(
        emb_t,
        res_W1,
        res_b1.reshape(N_UNITS, 1, HID),
        res_W2,
        res_b2.reshape(N_UNITS, 1, D),
        Wd.reshape(1, D),
        bd.reshape(1, 1),
    )
    return out


# speedup vs baseline: 4.7298x; 1.0758x over previous
import functools

import jax
import jax.numpy as jnp
from jax import lax
from jax.experimental import pallas as pl
from jax.experimental.pallas import tpu as pltpu
from jax.experimental.pallas import tpu_sc as plsc

N_FIELDS = 26
VOCAB = 100000
EMB = 16
BATCH = 4096
D = N_FIELDS * EMB
HID = 256
N_UNITS = 3

NC = 2
NS = 16
WIN = 1024
NWIN = 98
LAST_CW = 640
TAIL0 = 97 * WIN + LAST_CW
TAILW = VOCAB - TAIL0
NBK = 13
ACC_W = BATCH + 1
SENT_B = BATCH


def _sc_gather(tt, tail, keys2):
    mesh = plsc.VectorSubcoreMesh(core_axis_name="c", subcore_axis_name="s")

    @functools.partial(
        pl.kernel,
        out_type=jax.ShapeDtypeStruct((D, BATCH), jnp.float32),
        mesh=mesh,
        scratch_types=[
            pltpu.VMEM((2, EMB, WIN), jnp.float32),
            pltpu.VMEM((EMB, TAILW), jnp.float32),
            pltpu.VMEM((32, 128), jnp.int32),
            pltpu.VMEM((BATCH + 16,), jnp.int32),
            pltpu.VMEM((BATCH + 16,), jnp.int32),
            pltpu.VMEM((BATCH + 16,), jnp.int32),
            pltpu.VMEM((EMB, ACC_W), jnp.float32),
            pltpu.SemaphoreType.DMA,
        ],
        compiler_params=pltpu.CompilerParams(needs_layout_passes=False),
    )
    def gather_kernel(tt_hbm, tail_hbm, keys_hbm, out_hbm,
                      chunk_v, tail_v, keys_v, sup_v, bkt_v, hits_v, acc_v,
                      csem):
        f = lax.axis_index("s") * NC + lax.axis_index("c")
        row_iota = lax.iota(jnp.int32, 16)

        def compact(dst, kv, m, cnt):
            mi = m.astype(jnp.int32)
            incl = plsc.cumsum(mi)
            plsc.store_scatter(dst, [cnt + incl - mi], kv, mask=m)
            return cnt + incl[15]

        @pl.when(f < N_FIELDS)
        def _():
            fr = pl.multiple_of(f * 32, 8)
            fe = pl.multiple_of(f * EMB, 8)
            pltpu.sync_copy(keys_hbm.at[pl.ds(fr, 32)], keys_v)
            pltpu.sync_copy(tail_hbm.at[pl.ds(fe, EMB), :], tail_v)

            def fire(w, p):
                w0a = pl.multiple_of(w * WIN, 128)

                @pl.when(w < NWIN - 1)
                def _():
                    pltpu.async_copy(
                        tt_hbm.at[pl.ds(fe, EMB), pl.ds(w0a, WIN)],
                        chunk_v.at[p], csem,
                    )

                @pl.when(w == NWIN - 1)
                def _():
                    pltpu.async_copy(
                        tt_hbm.at[pl.ds(fe, EMB), pl.ds(w0a, LAST_CW)],
                        chunk_v.at[p].at[:, pl.ds(0, LAST_CW)], csem,
                    )

            def wait_chunk(w, p):
                @pl.when(w < NWIN - 1)
                def _():
                    pltpu.make_async_copy(
                        tt_hbm.at[pl.ds(fe, EMB), pl.ds(0, WIN)],
                        chunk_v.at[p], csem,
                    ).wait()

                @pl.when(w == NWIN - 1)
                def _():
                    pltpu.make_async_copy(
                        tt_hbm.at[pl.ds(fe, EMB), pl.ds(0, LAST_CW)],
                        chunk_v.at[p].at[:, pl.ds(0, LAST_CW)], csem,
                    ).wait()

            fire(0, 0)

            def bucket_body(bk, nsp):
                blo = bk << 26
                bhi = (bk + 1) << 26

                def bscan(g, cnt):
                    kv = plsc.load_gather(sup_v, [g * 16 + row_iota])
                    m = (kv >= blo) & (kv < bhi)
                    return compact(bkt_v, kv, m, cnt)

                nbk = lax.fori_loop(0, (nsp + 15) // 16, bscan, 0)
                plsc.store_scatter(
                    bkt_v, [nbk + row_iota],
                    jnp.full((16,), -1, jnp.int32),
                )

                def win_body(wi, _c1):
                    w = bk * 8 + wi

                    @pl.when(w < NWIN)
                    def _():
                        w0 = w * WIN
                        p = w % 2
                        chk = chunk_v.at[p]
                        wait_chunk(w, p)

                        @pl.when(w + 1 < NWIN)
                        def _():
                            fire(w + 1, 1 - p)

                        lo = w0 << 13
                        hi = (w0 + WIN) << 13

                        def wscan(g, cnt):
                            kv = plsc.load_gather(bkt_v, [g * 16 + row_iota])
                            m = (kv >= lo) & (kv < hi)
                            return compact(hits_v, kv, m, cnt)

                        ngb = (nbk + 15) // 16
                        cnt = lax.fori_loop(0, ngb, wscan, 0)
                        sent = jnp.full(
                            (16,), (w0 << 13) | SENT_B, dtype=jnp.int32
                        )
                        plsc.store_scatter(hits_v, [cnt + row_iota], sent)

                        def hit_group(j, _c2):
                            kv = plsc.load_gather(
                                hits_v, [j * 16 + row_iota]
                            )
                            for l in range(16):
                                key = kv[l]
                                v = key >> 13
                                b = key & 8191
                                col = v - w0
                                bcol = jnp.full((16,), b, jnp.int32)

                                @pl.when(v < TAIL0)
                                def _():
                                    cv = plsc.load_gather(
                                        chk,
                                        [row_iota,
                                         jnp.full((16,), col, jnp.int32)],
                                    )
                                    plsc.store_scatter(
                                        acc_v, [row_iota, bcol], cv
                                    )

                                @pl.when(v >= TAIL0)
                                def _():
                                    cv = plsc.load_gather(
                                        tail_v,
                                        [row_iota,
                                         jnp.full((16,), v - TAIL0,
                                                  jnp.int32)],
                                    )
                                    plsc.store_scatter(
                                        acc_v, [row_iota, bcol], cv
                                    )
                            return _c2

                        lax.fori_loop(0, (cnt + 15) // 16, hit_group, 0)

                    return _c1

                lax.fori_loop(0, 8, win_body, 0)

            def super_body(sp, _c):
                slo = sp << 28
                shi = (sp + 1) << 28

                def sscan(g, cnt):
                    for cc in range(8):
                        kv = keys_v[g, pl.ds(cc * 16, 16)]
                        m = (kv >= slo) & (kv < shi)
                        cnt = compact(sup_v, kv, m, cnt)
                    return cnt

                nsp = lax.fori_loop(0, 32, sscan, 0)
                plsc.store_scatter(
                    sup_v, [nsp + row_iota], jnp.full((16,), -1, jnp.int32)
                )

                def bq_body(bq, cc):
                    bucket_body(sp * 4 + bq, nsp)
                    return cc

                lax.fori_loop(0, 4, bq_body, 0)
                return _c

            lax.fori_loop(0, 4, super_body, 0)
            pltpu.sync_copy(
                acc_v.at[:, pl.ds(0, BATCH)], out_hbm.at[pl.ds(fe, EMB), :]
            )

    return gather_kernel(tt, tail, keys2)


def _mlp_body(x_ref, w1_ref, b1_ref, w2_ref, b2_ref, wd_ref, bd_ref, o_ref):
    r = x_ref[...].T
    for i in range(N_UNITS):
        h = jnp.dot(r, w1_ref[i], preferred_element_type=jnp.float32)
        h = jnp.maximum(h + b1_ref[i], 0.0)
        h = jnp.dot(h, w2_ref[i], preferred_element_type=jnp.float32)
        r = jnp.maximum(h + b2_ref[i] + r, 0.0)
    z = jnp.sum(r * wd_ref[...], axis=1, keepdims=True) + bd_ref[...]
    o_ref[...] = 1.0 / (1.0 + jnp.exp(-z))


def _mlp(emb_t, res_W1, res_b1, res_W2, res_b2, wd_row, bd11, block_b=512):
    grid = (BATCH // block_b,)
    return pl.pallas_call(
        _mlp_body,
        grid=grid,
        in_specs=[
            pl.BlockSpec((D, block_b), lambda i: (0, i)),
            pl.BlockSpec((N_UNITS, D, HID), lambda i: (0, 0, 0)),
            pl.BlockSpec((N_UNITS, 1, HID), lambda i: (0, 0, 0)),
            pl.BlockSpec((N_UNITS, HID, D), lambda i: (0, 0, 0)),
            pl.BlockSpec((N_UNITS, 1, D), lambda i: (0, 0, 0)),
            pl.BlockSpec((1, D), lambda i: (0, 0)),
            pl.BlockSpec((1, 1), lambda i: (0, 0)),
        ],
        out_specs=pl.BlockSpec((block_b, 1), lambda i: (i, 0)),
        out_shape=jax.ShapeDtypeStruct((BATCH, 1), jnp.float32),
    )(emb_t, res_W1, res_b1, res_W2, res_b2, wd_row, bd11)


def kernel(inputs, tables, res_W1, res_b1, res_W2, res_b2, Wd, bd):
    tt = tables.transpose(0, 2, 1).reshape(D, VOCAB)
    tail = lax.slice(tt, (0, TAIL0), (D, VOCAB))
    keys2 = (
        (inputs.astype(jnp.int32).T << 13)
        | jnp.arange(BATCH, dtype=jnp.int32)[None, :]
    ).reshape(832, 128)
    emb_t = _sc_gather(tt, tail, keys2)
    out = _mlp(
        emb_t,
        res_W1,
        res_b1.reshape(N_UNITS, 1, HID),
        res_W2,
        res_b2.reshape(N_UNITS, 1, D),
        Wd.reshape(1, D),
        bd.reshape(1, 1),
    )
    return out


# speedup vs baseline: 5.2084x; 1.1012x over previous
import functools

import jax
import jax.numpy as jnp
from jax import lax
from jax.experimental import pallas as pl
from jax.experimental.pallas import tpu as pltpu
from jax.experimental.pallas import tpu_sc as plsc

N_FIELDS = 26
VOCAB = 100000
EMB = 16
BATCH = 4096
D = N_FIELDS * EMB
HID = 256
N_UNITS = 3

NC = 2
NS = 16
WIN = 1024
NWIN = 98
LAST_CW = 640
TAIL0 = 97 * WIN + LAST_CW
TAILW = VOCAB - TAIL0
NBK = 13
ACC_W = BATCH + 1
SENT_B = BATCH


def _sc_gather(tt, tail, keys2):
    mesh = plsc.VectorSubcoreMesh(core_axis_name="c", subcore_axis_name="s")

    @functools.partial(
        pl.kernel,
        out_type=jax.ShapeDtypeStruct((D, BATCH), jnp.float32),
        mesh=mesh,
        scratch_types=[
            pltpu.VMEM((2, EMB, WIN), jnp.float32),
            pltpu.VMEM((EMB, TAILW), jnp.float32),
            pltpu.VMEM((32, 128), jnp.int32),
            pltpu.VMEM((BATCH + 16,), jnp.int32),
            pltpu.VMEM((BATCH + 16,), jnp.int32),
            pltpu.VMEM((BATCH + 16,), jnp.int32),
            pltpu.VMEM((EMB, ACC_W), jnp.float32),
            pltpu.SemaphoreType.DMA,
        ],
        compiler_params=pltpu.CompilerParams(needs_layout_passes=False),
    )
    def gather_kernel(tt_hbm, tail_hbm, keys_hbm, out_hbm,
                      chunk_v, tail_v, keys_v, sup_v, bkt_v, hits_v, acc_v,
                      csem):
        f = lax.axis_index("s") * NC + lax.axis_index("c")
        row_iota = lax.iota(jnp.int32, 16)

        def compact(dst, kv, m, cnt):
            mi = m.astype(jnp.int32)
            incl = plsc.cumsum(mi)
            plsc.store_scatter(dst, [cnt + incl - mi], kv, mask=m)
            return cnt + incl[15]

        @pl.when(f < N_FIELDS)
        def _():
            fr = pl.multiple_of(f * 32, 8)
            fe = pl.multiple_of(f * EMB, 8)
            pltpu.sync_copy(keys_hbm.at[pl.ds(fr, 32)], keys_v)
            pltpu.sync_copy(tail_hbm.at[pl.ds(fe, EMB), :], tail_v)

            def fire(w, p):
                w0a = pl.multiple_of(w * WIN, 128)

                @pl.when(w < NWIN - 1)
                def _():
                    pltpu.async_copy(
                        tt_hbm.at[pl.ds(fe, EMB), pl.ds(w0a, WIN)],
                        chunk_v.at[p], csem,
                    )

                @pl.when(w == NWIN - 1)
                def _():
                    pltpu.async_copy(
                        tt_hbm.at[pl.ds(fe, EMB), pl.ds(w0a, LAST_CW)],
                        chunk_v.at[p].at[:, pl.ds(0, LAST_CW)], csem,
                    )

            def wait_chunk(w, p):
                @pl.when(w < NWIN - 1)
                def _():
                    pltpu.make_async_copy(
                        tt_hbm.at[pl.ds(fe, EMB), pl.ds(0, WIN)],
                        chunk_v.at[p], csem,
                    ).wait()

                @pl.when(w == NWIN - 1)
                def _():
                    pltpu.make_async_copy(
                        tt_hbm.at[pl.ds(fe, EMB), pl.ds(0, LAST_CW)],
                        chunk_v.at[p].at[:, pl.ds(0, LAST_CW)], csem,
                    ).wait()

            fire(0, 0)

            def bucket_body(bk, nsp):
                blo = bk << 26
                bhi = (bk + 1) << 26

                def bscan(g, cnt):
                    kv = plsc.load_gather(sup_v, [g * 16 + row_iota])
                    m = (kv >= blo) & (kv < bhi)
                    return compact(bkt_v, kv, m, cnt)

                nbk = lax.fori_loop(0, (nsp + 15) // 16, bscan, 0)
                plsc.store_scatter(
                    bkt_v, [nbk + row_iota],
                    jnp.full((16,), -1, jnp.int32),
                )

                def win_body(wi, _c1):
                    w = bk * 8 + wi

                    @pl.when(w < NWIN)
                    def _():
                        w0 = w * WIN
                        p = w % 2
                        chk = chunk_v.at[p]
                        wait_chunk(w, p)

                        @pl.when(w + 1 < NWIN)
                        def _():
                            fire(w + 1, 1 - p)

                        lo = w0 << 13
                        hi = (w0 + WIN) << 13

                        def wscan(g, cnt):
                            kv = plsc.load_gather(bkt_v, [g * 16 + row_iota])
                            m = (kv >= lo) & (kv < hi)
                            return compact(hits_v, kv, m, cnt)

                        ngb = (nbk + 15) // 16
                        cnt = lax.fori_loop(0, ngb, wscan, 0)
                        sent = jnp.full(
                            (16,), (w0 << 13) | SENT_B, dtype=jnp.int32
                        )
                        plsc.store_scatter(hits_v, [cnt + row_iota], sent)

                        def hit_group(j, _c2):
                            kv = plsc.load_gather(
                                hits_v, [j * 16 + row_iota]
                            )

                            @pl.when(w < NWIN - 1)
                            def _():
                                for l in range(16):
                                    key = kv[l]
                                    col = (key >> 13) - w0
                                    bcol = jnp.full(
                                        (16,), key & 8191, jnp.int32
                                    )
                                    cv = plsc.load_gather(
                                        chk,
                                        [row_iota,
                                         jnp.full((16,), col, jnp.int32)],
                                    )
                                    plsc.store_scatter(
                                        acc_v, [row_iota, bcol], cv
                                    )

                            @pl.when(w == NWIN - 1)
                            def _():
                                for l in range(16):
                                    key = kv[l]
                                    v = key >> 13
                                    b = key & 8191
                                    col = v - w0
                                    bcol = jnp.full((16,), b, jnp.int32)

                                    @pl.when(v < TAIL0)
                                    def _():
                                        cv = plsc.load_gather(
                                            chk,
                                            [row_iota,
                                             jnp.full((16,), col,
                                                      jnp.int32)],
                                        )
                                        plsc.store_scatter(
                                            acc_v, [row_iota, bcol], cv
                                        )

                                    @pl.when(v >= TAIL0)
                                    def _():
                                        cv = plsc.load_gather(
                                            tail_v,
                                            [row_iota,
                                             jnp.full((16,), v - TAIL0,
                                                      jnp.int32)],
                                        )
                                        plsc.store_scatter(
                                            acc_v, [row_iota, bcol], cv
                                        )
                            return _c2

                        lax.fori_loop(0, (cnt + 15) // 16, hit_group, 0)

                    return _c1

                lax.fori_loop(0, 8, win_body, 0)

            def super_body(sp, _c):
                slo = sp << 28
                shi = (sp + 1) << 28

                def sscan(g, cnt):
                    for cc in range(8):
                        kv = keys_v[g, pl.ds(cc * 16, 16)]
                        m = (kv >= slo) & (kv < shi)
                        cnt = compact(sup_v, kv, m, cnt)
                    return cnt

                nsp = lax.fori_loop(0, 32, sscan, 0)
                plsc.store_scatter(
                    sup_v, [nsp + row_iota], jnp.full((16,), -1, jnp.int32)
                )

                def bq_body(bq, cc):
                    bucket_body(sp * 4 + bq, nsp)
                    return cc

                lax.fori_loop(0, 4, bq_body, 0)
                return _c

            lax.fori_loop(0, 4, super_body, 0)
            pltpu.sync_copy(
                acc_v.at[:, pl.ds(0, BATCH)], out_hbm.at[pl.ds(fe, EMB), :]
            )

    return gather_kernel(tt, tail, keys2)


def _mlp_body(x_ref, w1_ref, b1_ref, w2_ref, b2_ref, wd_ref, bd_ref, o_ref):
    r = x_ref[...].T
    for i in range(N_UNITS):
        h = jnp.dot(r, w1_ref[i], preferred_element_type=jnp.float32)
        h = jnp.maximum(h + b1_ref[i], 0.0)
        h = jnp.dot(h, w2_ref[i], preferred_element_type=jnp.float32)
        r = jnp.maximum(h + b2_ref[i] + r, 0.0)
    z = jnp.sum(r * wd_ref[...], axis=1, keepdims=True) + bd_ref[...]
    o_ref[...] = 1.0 / (1.0 + jnp.exp(-z))


def _mlp(emb_t, res_W1, res_b1, res_W2, res_b2, wd_row, bd11, block_b=512):
    grid = (BATCH // block_b,)
    return pl.pallas_call(
        _mlp_body,
        grid=grid,
        in_specs=[
            pl.BlockSpec((D, block_b), lambda i: (0, i)),
            pl.BlockSpec((N_UNITS, D, HID), lambda i: (0, 0, 0)),
            pl.BlockSpec((N_UNITS, 1, HID), lambda i: (0, 0, 0)),
            pl.BlockSpec((N_UNITS, HID, D), lambda i: (0, 0, 0)),
            pl.BlockSpec((N_UNITS, 1, D), lambda i: (0, 0, 0)),
            pl.BlockSpec((1, D), lambda i: (0, 0)),
            pl.BlockSpec((1, 1), lambda i: (0, 0)),
        ],
        out_specs=pl.BlockSpec((block_b, 1), lambda i: (i, 0)),
        out_shape=jax.ShapeDtypeStruct((BATCH, 1), jnp.float32),
    )(emb_t, res_W1, res_b1, res_W2, res_b2, wd_row, bd11)


def kernel(inputs, tables, res_W1, res_b1, res_W2, res_b2, Wd, bd):
    tt = tables.transpose(0, 2, 1).reshape(D, VOCAB)
    tail = lax.slice(tt, (0, TAIL0), (D, VOCAB))
    keys2 = (
        (inputs.astype(jnp.int32).T << 13)
        | jnp.arange(BATCH, dtype=jnp.int32)[None, :]
    ).reshape(832, 128)
    emb_t = _sc_gather(tt, tail, keys2)
    out = _mlp(
        emb_t,
        res_W1,
        res_b1.reshape(N_UNITS, 1, HID),
        res_W2,
        res_b2.reshape(N_UNITS, 1, D),
        Wd.reshape(1, D),
        bd.reshape(1, 1),
    )
    return out
